# baseline (speedup 1.0000x reference)
import jax
import jax.numpy as jnp
from jax import lax
from jax.experimental import pallas as pl
from jax.experimental.pallas import tpu as pltpu
from jax.experimental.pallas import tpu_sc as plsc

Q = 1024
D = 64
N = 100000
K = 16
BN = 4096
NB = 25
NPAD = NB * BN
C = 128
CPB = BN // C
CHUNKS = NB * CPB
NREAL = -(-N // C)
T = 20

SC_CORES = 2
SC_SUBCORES = 16
SC_WORKERS = SC_CORES * SC_SUBCORES
IDXW = 128


def _l2_normalize(x, eps=1e-12):
    n = jnp.linalg.norm(x, ord=2, axis=1, keepdims=True)
    return x / jnp.maximum(n, eps)


def _sims_body(qn_ref, rn_ref, sims_ref, m_ref):
    b = pl.program_id(0)
    s = lax.dot_general(
        qn_ref[...], rn_ref[...],
        dimension_numbers=(((1,), (1,)), ((), ())),
        preferred_element_type=jnp.float32,
    )
    s3 = s.reshape(Q, CPB, C)
    sims_ref[...] = s3
    m_ref[...] = jnp.max(s3, axis=2)[None]


_sims_call = pl.pallas_call(
    _sims_body,
    grid=(NB,),
    in_specs=[
        pl.BlockSpec((Q, D), lambda b: (0, 0)),
        pl.BlockSpec((BN, D), lambda b: (b, 0)),
    ],
    out_specs=[
        pl.BlockSpec((Q, CPB, C), lambda b: (0, b, 0)),
        pl.BlockSpec((1, Q, CPB), lambda b: (b, 0, 0)),
    ],
    out_shape=[
        jax.ShapeDtypeStruct((Q, CHUNKS, C), jnp.float32),
        jax.ShapeDtypeStruct((NB, Q, CPB), jnp.float32),
    ],
)


_QT = 512


def _chunks_body(m_ref, gid_ref):
    i = pl.program_id(0)
    vals = m_ref[...]
    gcid = lax.broadcasted_iota(jnp.int32, (_QT, CHUNKS), 1)
    vals = jnp.where(gcid < NREAL, vals, -jnp.inf)
    cols = []
    for _ in range(T):
        mx = jnp.max(vals, axis=1, keepdims=True)
        sel = jnp.min(jnp.where(vals == mx, gcid, jnp.int32(2**30)),
                      axis=1, keepdims=True)
        cols.append(sel)
        vals = jnp.where(gcid == sel, -jnp.inf, vals)
    qrow = i * _QT + lax.broadcasted_iota(jnp.int32, (_QT, T), 0)
    gid_ref[...] = jnp.concatenate(cols, axis=1) + qrow * CHUNKS


_chunks_call = pl.pallas_call(
    _chunks_body,
    grid=(Q // _QT,),
    in_specs=[pl.BlockSpec((_QT, CHUNKS), lambda i: (i, 0))],
    out_specs=pl.BlockSpec((_QT, T), lambda i: (i, 0)),
    out_shape=jax.ShapeDtypeStruct((Q, T), jnp.int32),
)


import functools


@functools.lru_cache(maxsize=None)
def _make_sc_gather(rows, d, nseg, tc_tiling=True):
    per_w = nseg * IDXW
    assert per_w * SC_WORKERS == rows

    def body(table_hbm, idx_hbm, out_hbm, idx_v, rows_v, sem):
        wid = lax.axis_index("s") * SC_CORES + lax.axis_index("c")
        pltpu.sync_copy(idx_hbm.at[wid], idx_v)
        copies = []
        for j in range(nseg):
            copies.append(pltpu.async_copy(
                table_hbm.at[idx_v.at[j]],
                rows_v.at[pl.ds(j * IDXW, IDXW)],
                sem,
            ))
        for cp in copies:
            cp.wait()
        pltpu.sync_copy(rows_v, out_hbm.at[pl.ds(wid * per_w, per_w)])

    mesh = plsc.VectorSubcoreMesh(
        core_axis_name="c", subcore_axis_name="s",
        num_cores=SC_CORES, num_subcores=SC_SUBCORES,
    )
    return pl.kernel(
        body,
        out_type=jax.ShapeDtypeStruct((rows, d), jnp.float32),
        mesh=mesh,
        scratch_types=[
            pltpu.VMEM((nseg, IDXW), jnp.int32),
            pltpu.VMEM((per_w, d), jnp.float32),
            pltpu.SemaphoreType.DMA,
        ],
        compiler_params=pltpu.CompilerParams(use_tc_tiling_on_sc=tc_tiling),
    )


def _gather_cand(table, idx):
    return _make_sc_gather(Q * T, C, (Q * T) // (SC_WORKERS * IDXW))(table, idx)


def _gather_img(table, idx):
    return _make_sc_gather(Q * K, D, (Q * K) // (SC_WORKERS * IDXW),
                           tc_tiling=False)(table, idx)


def _topk_body(cand_ref, gid_ref, out_ref):
    i = pl.program_id(0)
    vals = cand_ref[...]
    q = i * _QT + lax.broadcasted_iota(jnp.int32, (_QT, T), 0)
    chunk = gid_ref[...] - q * CHUNKS
    lane = lax.broadcasted_iota(jnp.int32, (_QT, T, C), 2)
    ridx = (chunk[:, :, None] * C + lane).reshape(_QT, T * C)
    vals = jnp.where(ridx < N, vals, -jnp.inf)
    cols = []
    for _ in range(K):
        mx = jnp.max(vals, axis=1, keepdims=True)
        sel = jnp.min(jnp.where(vals == mx, ridx, jnp.int32(2**30)),
                      axis=1, keepdims=True)
        cols.append(sel)
        vals = jnp.where(ridx == sel, -jnp.inf, vals)
    out_ref[...] = jnp.concatenate(cols, axis=1)


_topk_call = pl.pallas_call(
    _topk_body,
    grid=(Q // _QT,),
    in_specs=[
        pl.BlockSpec((_QT, T * C), lambda i: (i, 0)),
        pl.BlockSpec((_QT, T), lambda i: (i, 0)),
    ],
    out_specs=pl.BlockSpec((_QT, K), lambda i: (i, 0)),
    out_shape=jax.ShapeDtypeStruct((Q, K), jnp.int32),
)


def _half_body(v_ref, fidx_ref, out_ref):
    v = v_ref[...]
    p = (fidx_ref[...] & 1) == 1
    out_ref[...] = jnp.where(p, v[:, :, D:], v[:, :, :D])


_half_call = pl.pallas_call(
    _half_body,
    grid=(Q // _QT,),
    in_specs=[
        pl.BlockSpec((_QT, K, 2 * D), lambda i: (i, 0, 0)),
        pl.BlockSpec((_QT, K, 1), lambda i: (i, 0, 0)),
    ],
    out_specs=pl.BlockSpec((_QT, K, D), lambda i: (i, 0, 0)),
    out_shape=jax.ShapeDtypeStruct((Q, K, D), jnp.float32),
)


def kernel(query_features, reference_features, reference_images):
    qn = _l2_normalize(query_features)
    rn_pad = jnp.pad(_l2_normalize(reference_features), ((0, NPAD - N), (0, 0)))

    sims3, m = _sims_call(qn, rn_pad)
    mt = jnp.transpose(m, (1, 0, 2)).reshape(Q, CHUNKS)
    gid = _chunks_call(mt)
    table = sims3.reshape(Q * CHUNKS, C)
    cand = _gather_cand(table, gid.reshape(SC_WORKERS, -1, IDXW))
    fidx = _topk_call(cand.reshape(Q, T * C), gid)
    img = _gather_img(reference_images,
                      fidx.reshape(SC_WORKERS, -1, IDXW))
    return img.reshape(Q, K, D)


# speedup vs baseline: 1.0201x; 1.0201x over previous
import jax
import jax.numpy as jnp
from jax import lax
from jax.experimental import pallas as pl
from jax.experimental.pallas import tpu as pltpu
from jax.experimental.pallas import tpu_sc as plsc

Q = 1024
D = 64
N = 100000
K = 16
BN = 4096
NB = 25
NPAD = NB * BN
C = 128
CPB = BN // C
CHUNKS = NB * CPB
NREAL = -(-N // C)
T = 20

SC_CORES = 2
SC_SUBCORES = 16
SC_WORKERS = SC_CORES * SC_SUBCORES
IDXW = 128


def _l2_normalize(x, eps=1e-12):
    n = jnp.linalg.norm(x, ord=2, axis=1, keepdims=True)
    return x / jnp.maximum(n, eps)


def _sims_body(qn_ref, rn_ref, sims_ref, m_ref):
    b = pl.program_id(0)
    s = lax.dot_general(
        qn_ref[...], rn_ref[...],
        dimension_numbers=(((1,), (1,)), ((), ())),
        preferred_element_type=jnp.float32,
    )
    s3 = s.reshape(Q, CPB, C)
    sims_ref[...] = s3
    m_ref[...] = jnp.max(s3, axis=2)[None]


_sims_call = pl.pallas_call(
    _sims_body,
    grid=(NB,),
    in_specs=[
        pl.BlockSpec((Q, D), lambda b: (0, 0)),
        pl.BlockSpec((BN, D), lambda b: (b, 0)),
    ],
    out_specs=[
        pl.BlockSpec((Q, CPB, C), lambda b: (0, b, 0)),
        pl.BlockSpec((1, Q, CPB), lambda b: (b, 0, 0)),
    ],
    out_shape=[
        jax.ShapeDtypeStruct((Q, CHUNKS, C), jnp.float32),
        jax.ShapeDtypeStruct((NB, Q, CPB), jnp.float32),
    ],
)


_QT = 256


def _chunks_body(m_ref, gid_ref):
    i = pl.program_id(0)
    vals = m_ref[...]
    gcid = lax.broadcasted_iota(jnp.int32, (_QT, CHUNKS), 1)
    vals = jnp.where(gcid < NREAL, vals, -jnp.inf)
    cols = []
    for _ in range(T):
        mx = jnp.max(vals, axis=1, keepdims=True)
        sel = jnp.min(jnp.where(vals == mx, gcid, jnp.int32(2**30)),
                      axis=1, keepdims=True)
        cols.append(sel)
        vals = jnp.where(gcid == sel, -jnp.inf, vals)
    qrow = i * _QT + lax.broadcasted_iota(jnp.int32, (_QT, T), 0)
    gid_ref[...] = jnp.concatenate(cols, axis=1) + qrow * CHUNKS


_chunks_call = pl.pallas_call(
    _chunks_body,
    grid=(Q // _QT,),
    in_specs=[pl.BlockSpec((_QT, CHUNKS), lambda i: (i, 0))],
    out_specs=pl.BlockSpec((_QT, T), lambda i: (i, 0)),
    out_shape=jax.ShapeDtypeStruct((Q, T), jnp.int32),
)


import functools


@functools.lru_cache(maxsize=None)
def _make_sc_gather(rows, d, nseg, tc_tiling=True):
    per_w = nseg * IDXW
    assert per_w * SC_WORKERS == rows

    def body(table_hbm, idx_hbm, out_hbm, idx_v, rows_v, sem):
        wid = lax.axis_index("s") * SC_CORES + lax.axis_index("c")
        pltpu.sync_copy(idx_hbm.at[wid], idx_v)
        copies = []
        for j in range(nseg):
            copies.append(pltpu.async_copy(
                table_hbm.at[idx_v.at[j]],
                rows_v.at[pl.ds(j * IDXW, IDXW)],
                sem,
            ))
        for cp in copies:
            cp.wait()
        pltpu.sync_copy(rows_v, out_hbm.at[pl.ds(wid * per_w, per_w)])

    mesh = plsc.VectorSubcoreMesh(
        core_axis_name="c", subcore_axis_name="s",
        num_cores=SC_CORES, num_subcores=SC_SUBCORES,
    )
    return pl.kernel(
        body,
        out_type=jax.ShapeDtypeStruct((rows, d), jnp.float32),
        mesh=mesh,
        scratch_types=[
            pltpu.VMEM((nseg, IDXW), jnp.int32),
            pltpu.VMEM((per_w, d), jnp.float32),
            pltpu.SemaphoreType.DMA,
        ],
        compiler_params=pltpu.CompilerParams(use_tc_tiling_on_sc=tc_tiling),
    )


def _gather_cand(table, idx):
    return _make_sc_gather(Q * T, C, (Q * T) // (SC_WORKERS * IDXW))(table, idx)


def _gather_img(table, idx):
    return _make_sc_gather(Q * K, D, (Q * K) // (SC_WORKERS * IDXW),
                           tc_tiling=False)(table, idx)


def _topk_body(cand_ref, gid_ref, out_ref):
    i = pl.program_id(0)
    vals = cand_ref[...]
    q = i * _QT + lax.broadcasted_iota(jnp.int32, (_QT, T), 0)
    chunk = gid_ref[...] - q * CHUNKS
    lane = lax.broadcasted_iota(jnp.int32, (_QT, T, C), 2)
    ridx = (chunk[:, :, None] * C + lane).reshape(_QT, T * C)
    vals = jnp.where(ridx < N, vals, -jnp.inf)
    cols = []
    for _ in range(K):
        mx = jnp.max(vals, axis=1, keepdims=True)
        sel = jnp.min(jnp.where(vals == mx, ridx, jnp.int32(2**30)),
                      axis=1, keepdims=True)
        cols.append(sel)
        vals = jnp.where(ridx == sel, -jnp.inf, vals)
    out_ref[...] = jnp.concatenate(cols, axis=1)


_topk_call = pl.pallas_call(
    _topk_body,
    grid=(Q // _QT,),
    in_specs=[
        pl.BlockSpec((_QT, T * C), lambda i: (i, 0)),
        pl.BlockSpec((_QT, T), lambda i: (i, 0)),
    ],
    out_specs=pl.BlockSpec((_QT, K), lambda i: (i, 0)),
    out_shape=jax.ShapeDtypeStruct((Q, K), jnp.int32),
)


def _half_body(v_ref, fidx_ref, out_ref):
    v = v_ref[...]
    p = (fidx_ref[...] & 1) == 1
    out_ref[...] = jnp.where(p, v[:, :, D:], v[:, :, :D])


_half_call = pl.pallas_call(
    _half_body,
    grid=(Q // _QT,),
    in_specs=[
        pl.BlockSpec((_QT, K, 2 * D), lambda i: (i, 0, 0)),
        pl.BlockSpec((_QT, K, 1), lambda i: (i, 0, 0)),
    ],
    out_specs=pl.BlockSpec((_QT, K, D), lambda i: (i, 0, 0)),
    out_shape=jax.ShapeDtypeStruct((Q, K, D), jnp.float32),
)


def kernel(query_features, reference_features, reference_images):
    qn = _l2_normalize(query_features)
    rn_pad = jnp.pad(_l2_normalize(reference_features), ((0, NPAD - N), (0, 0)))

    sims3, m = _sims_call(qn, rn_pad)
    mt = jnp.transpose(m, (1, 0, 2)).reshape(Q, CHUNKS)
    gid = _chunks_call(mt)
    table = sims3.reshape(Q * CHUNKS, C)
    cand = _gather_cand(table, gid.reshape(SC_WORKERS, -1, IDXW))
    fidx = _topk_call(cand.reshape(Q, T * C), gid)
    img = _gather_img(reference_images,
                      fidx.reshape(SC_WORKERS, -1, IDXW))
    return img.reshape(Q, K, D)
